# Initial kernel scaffold; baseline (speedup 1.0000x reference)
import jax
import jax.numpy as jnp
from jax.experimental import pallas as pl


def kernel(input_e, category, tag, chapter, test, response, W_exe, W_cat, W_tag, W_chap, W_test, W_pos, W_resp):
    raise NotImplementedError("write your pallas kernel here")


# speedup vs baseline: 9.5435x; 9.5435x over previous
import functools

import jax
import jax.numpy as jnp
from jax import lax
from jax.experimental import pallas as pl
from jax.experimental.pallas import tpu as pltpu
from jax.experimental.pallas import tpu_sc as plsc

D = 64
L = 199
NW = 32
CH = 128


def _resppos_body(resp_ref, pos_ref, out_ref):
    out_ref[...] = resp_ref[...][:, None, :] + pos_ref[...][None, :, :]


def _build_resppos(w_resp, w_pos):
    out = pl.pallas_call(
        _resppos_body,
        out_shape=jax.ShapeDtypeStruct((4, L, D), jnp.float32),
    )(w_resp, w_pos)
    return out.reshape(4 * L, D)


def _sc_body(rows_per_w, idx_hbm, w_exe, w_cat, w_tag, w_rp, out_hbm,
             idx_v, idxrp_v, buf_e, buf_c, buf_t, buf_rp, sem):
    wid = lax.axis_index("s") * 2 + lax.axis_index("c")
    base = wid * rows_per_w
    nchunk = rows_per_w // CH

    def chunk(k, _):
        row0 = base + k * CH
        pltpu.sync_copy(idx_hbm.at[:, pl.ds(row0, CH)], idx_v)
        off = k * CH
        for s in range(CH // 16):
            j16 = lax.iota(jnp.int32, 16) + (off + s * 16)
            rv = idx_v[3, pl.ds(s * 16, 16)]
            idxrp_v[pl.ds(s * 16, 16)] = rv * L + j16 % L
        d1 = pltpu.async_copy(w_exe.at[idx_v.at[0]], buf_e, sem)
        d2 = pltpu.async_copy(w_cat.at[idx_v.at[1]], buf_c, sem)
        d3 = pltpu.async_copy(w_tag.at[idx_v.at[2]], buf_t, sem)
        d4 = pltpu.async_copy(w_rp.at[idxrp_v], buf_rp, sem)
        d1.wait()
        d2.wait()
        d3.wait()
        d4.wait()

        def red(j, _):
            for r in range(2):
                i = j * 2 + r
                for s in range(D // 16):
                    sl = pl.ds(s * 16, 16)
                    buf_e[i, sl] = ((buf_e[i, sl] + buf_c[i, sl])
                                    + (buf_t[i, sl] + buf_rp[i, sl]))
            return _

        lax.fori_loop(0, CH // 2, red, None)
        pltpu.sync_copy(buf_e, out_hbm.at[pl.ds(row0, CH)])
        return _

    lax.fori_loop(0, nchunk, chunk, None)


def kernel(input_e, category, tag, chapter, test, response,
           W_exe, W_cat, W_tag, W_chap, W_test, W_pos, W_resp):
    del chapter, test, W_chap, W_test
    B, Lc = input_e.shape
    n = B * Lc
    rows_per_w = n // NW

    idx_all = jnp.stack([
        input_e.reshape(-1).astype(jnp.int32),
        category.reshape(-1).astype(jnp.int32),
        tag.reshape(-1).astype(jnp.int32),
        response.reshape(-1).astype(jnp.int32),
    ])
    w_rp = _build_resppos(W_resp.astype(jnp.float32),
                          W_pos[:L].astype(jnp.float32))

    mesh = plsc.VectorSubcoreMesh(core_axis_name="c", subcore_axis_name="s")
    out = pl.kernel(
        functools.partial(_sc_body, rows_per_w),
        out_type=jax.ShapeDtypeStruct((n, D), jnp.float32),
        mesh=mesh,
        compiler_params=pltpu.CompilerParams(use_tc_tiling_on_sc=False),
        scratch_types=[
            pltpu.VMEM((4, CH), jnp.int32),
            pltpu.VMEM((CH,), jnp.int32),
            pltpu.VMEM((CH, D), jnp.float32),
            pltpu.VMEM((CH, D), jnp.float32),
            pltpu.VMEM((CH, D), jnp.float32),
            pltpu.VMEM((CH, D), jnp.float32),
            pltpu.SemaphoreType.DMA,
        ],
    )(idx_all, W_exe, W_cat, W_tag, w_rp)
    return out.reshape(B, Lc, D)


# speedup vs baseline: 12.6081x; 1.3211x over previous
import functools

import jax
import jax.numpy as jnp
from jax import lax
from jax.experimental import pallas as pl
from jax.experimental.pallas import tpu as pltpu
from jax.experimental.pallas import tpu_sc as plsc

D = 64
L = 199
NW = 32
CH = 128


def _resppos_body(resp_ref, pos_ref, out_ref):
    out_ref[...] = resp_ref[...][:, None, :] + pos_ref[...][None, :, :]


def _build_resppos(w_resp, w_pos):
    out = pl.pallas_call(
        _resppos_body,
        out_shape=jax.ShapeDtypeStruct((4, L, D), jnp.float32),
    )(w_resp, w_pos)
    return out.reshape(4 * L, D)


def _sc_body(rows_per_w, idx_hbm, w_exe, w_cat, w_tag, w_rp, out_hbm,
             idx_v0, idx_v1, idxrp_v0, idxrp_v1,
             be0, bc0, bt0, brp0, be1, bc1, bt1, brp1,
             sg0, sg1, si0, si1):
    wid = lax.axis_index("s") * 2 + lax.axis_index("c")
    base = wid * rows_per_w
    nchunk = rows_per_w // CH
    npair = (nchunk - 1) // 2

    sets = [
        (idx_v0, idxrp_v0, be0, bc0, bt0, brp0, sg0, si0),
        (idx_v1, idxrp_v1, be1, bc1, bt1, brp1, sg1, si1),
    ]

    def idx_desc(k, st):
        return pltpu.make_async_copy(
            idx_hbm.at[:, pl.ds(base + k * CH, CH)], st[0], st[7])

    def compute_rp(k, st):
        idx_v, idxrp_v = st[0], st[1]
        off = k * CH
        for s in range(CH // 16):
            j16 = lax.iota(jnp.int32, 16) + (off + s * 16)
            rv = idx_v[3, pl.ds(s * 16, 16)]
            idxrp_v[pl.ds(s * 16, 16)] = rv * L + j16 % L

    def gather_descs(st):
        idx_v, idxrp_v, be, bc, bt, brp, sg = st[:7]
        return (pltpu.make_async_copy(w_exe.at[idx_v.at[0]], be, sg),
                pltpu.make_async_copy(w_cat.at[idx_v.at[1]], bc, sg),
                pltpu.make_async_copy(w_tag.at[idx_v.at[2]], bt, sg),
                pltpu.make_async_copy(w_rp.at[idxrp_v], brp, sg))

    def fire_gathers(st):
        for d in gather_descs(st):
            d.start()

    def wait_gathers(st):
        for d in gather_descs(st):
            d.wait()

    def reduce_store(k, st):
        be, bc, bt, brp = st[2], st[3], st[4], st[5]

        def red(j, _):
            for r in range(2):
                i = j * 2 + r
                for s in range(D // 16):
                    sl = pl.ds(s * 16, 16)
                    be[i, sl] = (be[i, sl] + bc[i, sl]) + (bt[i, sl] + brp[i, sl])
            return _

        lax.fori_loop(0, CH // 2, red, None)
        pltpu.sync_copy(be, out_hbm.at[pl.ds(base + k * CH, CH)])

    pltpu.sync_copy(idx_hbm.at[:, pl.ds(base, CH)], idx_v0)
    compute_rp(0, sets[0])
    fire_gathers(sets[0])
    idx_desc(1, sets[1]).start()

    def pair(p, _):
        kb = 2 * p + 1
        kc = 2 * p + 2
        idx_desc(kb, sets[1]).wait()
        compute_rp(kb, sets[1])
        fire_gathers(sets[1])
        wait_gathers(sets[0])
        idx_desc(kc, sets[0]).start()
        reduce_store(2 * p, sets[0])
        idx_desc(kc, sets[0]).wait()
        compute_rp(kc, sets[0])
        fire_gathers(sets[0])
        wait_gathers(sets[1])
        pl.when(p < npair - 1)(lambda: idx_desc(2 * p + 3, sets[1]).start())
        reduce_store(kb, sets[1])
        return _

    lax.fori_loop(0, npair, pair, None)
    wait_gathers(sets[0])
    reduce_store(nchunk - 1, sets[0])


def kernel(input_e, category, tag, chapter, test, response,
           W_exe, W_cat, W_tag, W_chap, W_test, W_pos, W_resp):
    del chapter, test, W_chap, W_test
    B, Lc = input_e.shape
    n = B * Lc
    rows_per_w = n // NW

    idx_all = jnp.stack([
        input_e.reshape(-1).astype(jnp.int32),
        category.reshape(-1).astype(jnp.int32),
        tag.reshape(-1).astype(jnp.int32),
        response.reshape(-1).astype(jnp.int32),
    ])
    w_rp = _build_resppos(W_resp.astype(jnp.float32),
                          W_pos[:L].astype(jnp.float32))

    mesh = plsc.VectorSubcoreMesh(core_axis_name="c", subcore_axis_name="s")
    out = pl.kernel(
        functools.partial(_sc_body, rows_per_w),
        out_type=jax.ShapeDtypeStruct((n, D), jnp.float32),
        mesh=mesh,
        compiler_params=pltpu.CompilerParams(use_tc_tiling_on_sc=False),
        scratch_types=(
            [pltpu.VMEM((4, CH), jnp.int32)] * 2
            + [pltpu.VMEM((CH,), jnp.int32)] * 2
            + [pltpu.VMEM((CH, D), jnp.float32)] * 8
            + [pltpu.SemaphoreType.DMA] * 4
        ),
    )(idx_all, W_exe, W_cat, W_tag, w_rp)
    return out.reshape(B, Lc, D)
